# baseline (speedup 1.0000x reference)
import jax
import jax.numpy as jnp
from jax.experimental import pallas as pl
from jax.experimental.pallas import tpu as pltpu

B, MAXLEN, LOC_LEN, EMB = 16, 128, 512, 32
SU, SL, TU, TL = 100.0, 0.0, 1000.0, 0.0

TT = MAXLEN
NSPLIT = 4
CH = TT // NSPLIT


def _body(traj_len_ref, ds_ref, vec_ref, tabs_ref, out_ref, scratch, sems):
    i = pl.program_id(0)
    slot = jax.lax.rem(i, 2)

    @pl.when(i >= 2)
    def _wait_prev():
        for k in range(NSPLIT):
            pltpu.make_async_copy(
                scratch.at[slot, pl.ds(k * CH, CH)],
                out_ref.at[0, pl.ds(k * CH, CH)],
                sems.at[slot, k],
            ).wait()

    tl_b = traj_len_ref[i]
    t_iota = jax.lax.broadcasted_iota(jnp.int32, (TT, 1, 1), 0)
    m = tl_b > t_iota

    p0 = tabs_ref[0, 0] + tabs_ref[2, 0]
    p1 = tabs_ref[0, 1] + tabs_ref[2, 1]
    q0 = (tabs_ref[3, 0] - tabs_ref[2, 0]) * (1.0 / (TU - TL))
    q1 = (tabs_ref[3, 1] - tabs_ref[2, 1]) * (1.0 / (TU - TL))
    r0 = (tabs_ref[1, 0] - tabs_ref[0, 0]) * (1.0 / (SU - SL))
    r1 = (tabs_ref[1, 1] - tabs_ref[0, 1]) * (1.0 / (SU - SL))

    p = jnp.where(m, p1, p0)
    q = jnp.where(m, q1, q0)
    r = jnp.where(m, r1, r0)

    dt = vec_ref[0]
    s = p + q * dt
    ds = ds_ref[0]
    scratch[slot] = s + r * ds

    for k in range(NSPLIT):
        pltpu.make_async_copy(
            scratch.at[slot, pl.ds(k * CH, CH)],
            out_ref.at[i, pl.ds(k * CH, CH)],
            sems.at[slot, k],
        ).start()

    @pl.when(i == B - 1)
    def _drain():
        for sl in range(2):
            for k in range(NSPLIT):
                pltpu.make_async_copy(
                    scratch.at[sl, pl.ds(k * CH, CH)],
                    out_ref.at[0, pl.ds(k * CH, CH)],
                    sems.at[sl, k],
                ).wait()


def kernel(traj_loc, mat2, vec, traj_len, emb_su, emb_sl, emb_tu, emb_tl):
    tabs = jnp.stack([emb_sl, emb_su, emb_tl, emb_tu])[..., None]
    grid = (B,)
    out = pl.pallas_call(
        _body,
        grid_spec=pltpu.PrefetchScalarGridSpec(
            num_scalar_prefetch=1,
            grid=grid,
            in_specs=[
                pl.BlockSpec((1, TT, 1, LOC_LEN), lambda b, tl: (b, 0, 0, 0)),
                pl.BlockSpec((1, TT, 1, 1), lambda b, tl: (b, 0, 0, 0)),
                pl.BlockSpec((4, 2, EMB, 1), lambda b, tl: (0, 0, 0, 0)),
            ],
            out_specs=pl.BlockSpec(memory_space=pl.ANY),
            scratch_shapes=[
                pltpu.VMEM((2, TT, EMB, LOC_LEN), jnp.float32),
                pltpu.SemaphoreType.DMA((2, NSPLIT)),
            ],
        ),
        out_shape=jax.ShapeDtypeStruct((B, MAXLEN, EMB, LOC_LEN), jnp.float32),
        compiler_params=pltpu.CompilerParams(
            dimension_semantics=("arbitrary",),
        ),
    )(
        traj_len.astype(jnp.int32),
        mat2[:, :, None, :],
        vec[:, :, None, None],
        tabs,
    )
    return jnp.swapaxes(out, 2, 3)


# speedup vs baseline: 1.0293x; 1.0293x over previous
import jax
import jax.numpy as jnp
from jax.experimental import pallas as pl
from jax.experimental.pallas import tpu as pltpu

B, MAXLEN, LOC_LEN, EMB = 16, 128, 512, 32
SU, SL, TU, TL = 100.0, 0.0, 1000.0, 0.0

TT = MAXLEN
NSPLIT = 4
CH = TT // NSPLIT


def _body(traj_len_ref, ds_ref, vec_ref, tabs_ref, out_ref, scratch, sems):
    i = pl.program_id(0)
    slot = jax.lax.rem(i, 2)

    @pl.when(i >= 2)
    def _wait_prev():
        for k in range(NSPLIT):
            pltpu.make_async_copy(
                scratch.at[slot, pl.ds(k * CH, CH)],
                out_ref.at[0, pl.ds(k * CH, CH)],
                sems.at[slot, k],
            ).wait()

    tl_b = traj_len_ref[i]
    t_iota = jax.lax.broadcasted_iota(jnp.int32, (TT, 1, 1), 0)
    m = tl_b > t_iota

    p0 = tabs_ref[0, 0] + tabs_ref[2, 0]
    p1 = tabs_ref[0, 1] + tabs_ref[2, 1]
    q0 = (tabs_ref[3, 0] - tabs_ref[2, 0]) * (1.0 / (TU - TL))
    q1 = (tabs_ref[3, 1] - tabs_ref[2, 1]) * (1.0 / (TU - TL))
    r0 = (tabs_ref[1, 0] - tabs_ref[0, 0]) * (1.0 / (SU - SL))
    r1 = (tabs_ref[1, 1] - tabs_ref[0, 1]) * (1.0 / (SU - SL))

    p = jnp.where(m, p1, p0)
    q = jnp.where(m, q1, q0)
    r = jnp.where(m, r1, r0)

    dt = vec_ref[0]
    s = p + q * dt

    for k in range(NSPLIT):
        lo, hi = k * CH, (k + 1) * CH
        ds_k = ds_ref[0, lo:hi]
        scratch[slot, lo:hi] = s[lo:hi] + r[lo:hi] * ds_k
        pltpu.make_async_copy(
            scratch.at[slot, lo:hi],
            out_ref.at[i, lo:hi],
            sems.at[slot, k],
        ).start()

    @pl.when(i == B - 1)
    def _drain():
        for sl in range(2):
            for k in range(NSPLIT):
                pltpu.make_async_copy(
                    scratch.at[sl, pl.ds(k * CH, CH)],
                    out_ref.at[0, pl.ds(k * CH, CH)],
                    sems.at[sl, k],
                ).wait()


def kernel(traj_loc, mat2, vec, traj_len, emb_su, emb_sl, emb_tu, emb_tl):
    tabs = jnp.stack([emb_sl, emb_su, emb_tl, emb_tu])[..., None]
    grid = (B,)
    out = pl.pallas_call(
        _body,
        grid_spec=pltpu.PrefetchScalarGridSpec(
            num_scalar_prefetch=1,
            grid=grid,
            in_specs=[
                pl.BlockSpec((1, TT, 1, LOC_LEN), lambda b, tl: (b, 0, 0, 0)),
                pl.BlockSpec((1, TT, 1, 1), lambda b, tl: (b, 0, 0, 0)),
                pl.BlockSpec((4, 2, EMB, 1), lambda b, tl: (0, 0, 0, 0)),
            ],
            out_specs=pl.BlockSpec(memory_space=pl.ANY),
            scratch_shapes=[
                pltpu.VMEM((2, TT, EMB, LOC_LEN), jnp.float32),
                pltpu.SemaphoreType.DMA((2, NSPLIT)),
            ],
        ),
        out_shape=jax.ShapeDtypeStruct((B, MAXLEN, EMB, LOC_LEN), jnp.float32),
        compiler_params=pltpu.CompilerParams(
            dimension_semantics=("arbitrary",),
        ),
    )(
        traj_len.astype(jnp.int32),
        mat2[:, :, None, :],
        vec[:, :, None, None],
        tabs,
    )
    return jnp.swapaxes(out, 2, 3)


# speedup vs baseline: 1.0337x; 1.0043x over previous
import jax
import jax.numpy as jnp
from jax.experimental import pallas as pl
from jax.experimental.pallas import tpu as pltpu

B, MAXLEN, LOC_LEN, EMB = 16, 128, 512, 32
SU, SL, TU, TL = 100.0, 0.0, 1000.0, 0.0

TT = MAXLEN
NSPLIT = 8
CH = TT // NSPLIT


def _body(traj_len_ref, ds_ref, vec_ref, tabs_ref, out_ref, scratch, sems):
    i = pl.program_id(0)
    slot = jax.lax.rem(i, 2)

    @pl.when(i >= 2)
    def _wait_prev():
        for k in range(NSPLIT):
            pltpu.make_async_copy(
                scratch.at[slot, pl.ds(k * CH, CH)],
                out_ref.at[0, pl.ds(k * CH, CH)],
                sems.at[slot, k],
            ).wait()

    tl_b = traj_len_ref[i]
    t_iota = jax.lax.broadcasted_iota(jnp.int32, (TT, 1, 1), 0)
    m = tl_b > t_iota

    p0 = tabs_ref[0, 0] + tabs_ref[2, 0]
    p1 = tabs_ref[0, 1] + tabs_ref[2, 1]
    q0 = (tabs_ref[3, 0] - tabs_ref[2, 0]) * (1.0 / (TU - TL))
    q1 = (tabs_ref[3, 1] - tabs_ref[2, 1]) * (1.0 / (TU - TL))
    r0 = (tabs_ref[1, 0] - tabs_ref[0, 0]) * (1.0 / (SU - SL))
    r1 = (tabs_ref[1, 1] - tabs_ref[0, 1]) * (1.0 / (SU - SL))

    p = jnp.where(m, p1, p0)
    q = jnp.where(m, q1, q0)
    r = jnp.where(m, r1, r0)

    dt = vec_ref[0]
    s = p + q * dt

    for k in range(NSPLIT):
        lo, hi = k * CH, (k + 1) * CH
        ds_k = ds_ref[0, lo:hi]
        scratch[slot, lo:hi] = s[lo:hi] + r[lo:hi] * ds_k
        pltpu.make_async_copy(
            scratch.at[slot, lo:hi],
            out_ref.at[i, lo:hi],
            sems.at[slot, k],
        ).start()

    @pl.when(i == B - 1)
    def _drain():
        for sl in range(2):
            for k in range(NSPLIT):
                pltpu.make_async_copy(
                    scratch.at[sl, pl.ds(k * CH, CH)],
                    out_ref.at[0, pl.ds(k * CH, CH)],
                    sems.at[sl, k],
                ).wait()


def kernel(traj_loc, mat2, vec, traj_len, emb_su, emb_sl, emb_tu, emb_tl):
    tabs = jnp.stack([emb_sl, emb_su, emb_tl, emb_tu])[..., None]
    grid = (B,)
    out = pl.pallas_call(
        _body,
        grid_spec=pltpu.PrefetchScalarGridSpec(
            num_scalar_prefetch=1,
            grid=grid,
            in_specs=[
                pl.BlockSpec((1, TT, 1, LOC_LEN), lambda b, tl: (b, 0, 0, 0)),
                pl.BlockSpec((1, TT, 1, 1), lambda b, tl: (b, 0, 0, 0)),
                pl.BlockSpec((4, 2, EMB, 1), lambda b, tl: (0, 0, 0, 0)),
            ],
            out_specs=pl.BlockSpec(memory_space=pl.ANY),
            scratch_shapes=[
                pltpu.VMEM((2, TT, EMB, LOC_LEN), jnp.float32),
                pltpu.SemaphoreType.DMA((2, NSPLIT)),
            ],
        ),
        out_shape=jax.ShapeDtypeStruct((B, MAXLEN, EMB, LOC_LEN), jnp.float32),
        compiler_params=pltpu.CompilerParams(
            dimension_semantics=("arbitrary",),
        ),
    )(
        traj_len.astype(jnp.int32),
        mat2[:, :, None, :],
        vec[:, :, None, None],
        tabs,
    )
    return jnp.swapaxes(out, 2, 3)
